# Initial kernel scaffold; baseline (speedup 1.0000x reference)
import jax
import jax.numpy as jnp
from jax.experimental import pallas as pl


def kernel(scores, bbox_frame, im_info, anchors):
    raise NotImplementedError("write your pallas kernel here")


# speedup vs baseline: 1.2345x; 1.2345x over previous
import jax
import jax.numpy as jnp
from jax import lax
from jax.experimental import pallas as pl

_TOPN = 2000
_PAD = 2048
_TMAX = 7.0


def _decode_body(anc_ref, dlt_ref, aux_ref, out_ref):
    anc = anc_ref[0]
    dlt = dlt_ref[0]
    aux = aux_ref[0]
    w = anc[3:4] - anc[0:1] + 1.0
    h = anc[4:5] - anc[1:2] + 1.0
    l = anc[5:6] - anc[2:3] + 1.0
    cx = anc[0:1] + 0.5 * w
    cy = anc[1:2] + 0.5 * h
    ct = anc[2:3] + 0.5 * l
    pcx = dlt[0:1] * w + cx
    pcy = dlt[1:2] * h + cy
    pct = dlt[2:3] * l + ct
    pw = jnp.exp(dlt[3:4]) * w
    ph = jnp.exp(dlt[4:5]) * h
    pln = jnp.exp(dlt[5:6]) * l
    bval = lax.convert_element_type(pl.program_id(0), jnp.float32)
    out_ref[0, 0:1, :] = jnp.zeros_like(w) + bval
    out_ref[0, 1:2, :] = jnp.clip(pcx - 0.5 * pw, 0.0, aux[1:2])
    out_ref[0, 2:3, :] = jnp.clip(pcy - 0.5 * ph, 0.0, aux[2:3])
    out_ref[0, 3:4, :] = jnp.clip(pct - 0.5 * pln, 0.0, aux[3:4])
    out_ref[0, 4:5, :] = jnp.clip(pcx + 0.5 * pw, 0.0, aux[4:5])
    out_ref[0, 5:6, :] = jnp.clip(pcy + 0.5 * ph, 0.0, aux[5:6])
    out_ref[0, 6:7, :] = jnp.clip(pct + 0.5 * pln, 0.0, aux[6:7])
    out_ref[0, 7:8, :] = aux[7:8]


def kernel(scores, bbox_frame, im_info, anchors):
    B, N, _ = scores.shape
    sc = scores[:, :, 1]
    val, idx = lax.top_k(sc, _TOPN)
    anc_g = jnp.take(anchors, idx.reshape(-1), axis=0).reshape(B, _TOPN, 6)
    dlt_g = jnp.take_along_axis(bbox_frame, idx[:, :, None], axis=1)

    def to_rows(x):
        x = jnp.pad(x, ((0, 0), (0, _PAD - _TOPN), (0, 0)))
        x = x.transpose(0, 2, 1)
        return jnp.pad(x, ((0, 0), (0, 2), (0, 0)))

    anc_t = to_rows(anc_g)
    dlt_t = to_rows(dlt_g)
    hi_x = im_info[:, 1] - 1.0
    hi_y = im_info[:, 0] - 1.0
    tmax = jnp.full_like(hi_x, _TMAX)
    zero = jnp.zeros_like(hi_x)
    bounds = jnp.stack([zero, hi_x, hi_y, tmax, hi_x, hi_y, tmax], axis=1)
    aux = jnp.concatenate(
        [
            jnp.broadcast_to(bounds[:, :, None], (B, 7, _PAD)),
            jnp.pad(val, ((0, 0), (0, _PAD - _TOPN)))[:, None, :],
        ],
        axis=1,
    )
    out = pl.pallas_call(
        _decode_body,
        grid=(B,),
        in_specs=[
            pl.BlockSpec((1, 8, _PAD), lambda b: (b, 0, 0)),
            pl.BlockSpec((1, 8, _PAD), lambda b: (b, 0, 0)),
            pl.BlockSpec((1, 8, _PAD), lambda b: (b, 0, 0)),
        ],
        out_specs=pl.BlockSpec((1, 8, _PAD), lambda b: (b, 0, 0)),
        out_shape=jax.ShapeDtypeStruct((B, 8, _PAD), jnp.float32),
    )(anc_t, dlt_t, aux)
    return out.transpose(0, 2, 1)[:, :_TOPN, :]
